# baseline (speedup 1.0000x reference)
import functools
import math

import jax
import jax.numpy as jnp
from jax import lax
from jax.experimental import pallas as pl
from jax.experimental.pallas import tpu as pltpu
from jax.experimental.pallas import tpu_sc as plsc

N = 50000
E = 800000
H = 64
HH = 32
NRBF = 64
NF = 128
NG = 64
CUTOFF = 6.0
GAMMA = 10.0

NSUB = 16
CH = 128
NCHUNKS = E // CH
CH_PER_SUB = NCHUNKS // NSUB
CH_EXTRA = NCHUNKS - CH_PER_SUB * NSUB
NPAD = 50176
ROWS_PER_SUB = NPAD // NSUB
ZROWS = 392

_f32 = jnp.float32
_i32 = jnp.int32


_TN = 5000
_TE = 6400


def _embed_body(an_ref, nf_ref, emb_ref, lo_ref, hi_ref):
    a = an_ref[...]
    oh = (a == lax.broadcasted_iota(_i32, (1, 128), 1)).astype(_f32)
    h = jnp.dot(oh, emb_ref[...], preferred_element_type=_f32) + nf_ref[...]
    lo_ref[...] = h[:, :HH]
    hi_ref[...] = h[:, HH:]


def _embed_call(an2, node_features, emb_pad):
    grid = N // _TN
    return pl.pallas_call(
        _embed_body,
        grid=(grid,),
        in_specs=[
            pl.BlockSpec((_TN, 1), lambda i: (i, 0)),
            pl.BlockSpec((_TN, H), lambda i: (i, 0)),
            pl.BlockSpec((128, H), lambda i: (0, 0)),
        ],
        out_specs=[
            pl.BlockSpec((_TN, HH), lambda i: (i, 0)),
            pl.BlockSpec((_TN, HH), lambda i: (i, 0)),
        ],
        out_shape=[
            jax.ShapeDtypeStruct((N, HH), _f32),
            jax.ShapeDtypeStruct((N, HH), _f32),
        ],
    )(an2, node_features, emb_pad)


def _filter_body(d_ref, w1_ref, b1_ref, w2_ref, b2_ref, w3_ref, b3_ref,
                 lo_ref, hi_ref):
    d = d_ref[...]
    centers = lax.broadcasted_iota(_i32, (1, NRBF), 1).astype(_f32) * (
        CUTOFF / (NRBF - 1))
    rbf = jnp.exp(-GAMMA * (d - centers) ** 2)
    cf = 0.5 * (jnp.cos(math.pi * d / CUTOFF) + 1.0)
    cf = cf * (d < CUTOFF).astype(_f32)
    rbf = rbf * cf
    bf16 = jnp.bfloat16
    x = jnp.maximum(jnp.dot(rbf.astype(bf16), w1_ref[...].astype(bf16),
                            preferred_element_type=_f32) + b1_ref[...], 0.0)
    x = jnp.maximum(jnp.dot(x.astype(bf16), w2_ref[...].astype(bf16),
                            preferred_element_type=_f32) + b2_ref[...], 0.0)
    w = jnp.dot(x.astype(bf16), w3_ref[...].astype(bf16),
                preferred_element_type=_f32) + b3_ref[...]
    lo_ref[...] = w[:, :HH]
    hi_ref[...] = w[:, HH:]


def _filter_call(d, blk):
    grid = E // _TE
    full = lambda shape: pl.BlockSpec(shape, lambda i: (0, 0))
    return pl.pallas_call(
        _filter_body,
        grid=(grid,),
        in_specs=[
            pl.BlockSpec((_TE, 1), lambda i: (i, 0)),
            full((NRBF, NF)), full((1, NF)),
            full((NF, NF)), full((1, NF)),
            full((NF, H)), full((1, H)),
        ],
        out_specs=[
            pl.BlockSpec((_TE, HH), lambda i: (i, 0)),
            pl.BlockSpec((_TE, HH), lambda i: (i, 0)),
        ],
        out_shape=[
            jax.ShapeDtypeStruct((E, HH), _f32),
            jax.ShapeDtypeStruct((E, HH), _f32),
        ],
    )(d, blk['fw1'], blk['fb1'].reshape(1, NF),
      blk['fw2'], blk['fb2'].reshape(1, NF),
      blk['fw3'], blk['fb3'].reshape(1, H))


def _update_body(hlo_ref, hhi_ref, alo_ref, ahi_ref, cnt_ref,
                 w1_ref, b1_ref, w2_ref, b2_ref, g_ref, b_ref,
                 olo_ref, ohi_ref):
    h = jnp.concatenate([hlo_ref[...], hhi_ref[...]], axis=1)
    rc = 1.0 / jnp.maximum(cnt_ref[...][:, :1], 1.0)
    agg = jnp.concatenate([alo_ref[...], ahi_ref[...]], axis=1) * rc
    comb = jnp.concatenate([h, agg], axis=1)
    x = jnp.maximum(jnp.dot(comb, w1_ref[...], preferred_element_type=_f32)
                    + b1_ref[...], 0.0)
    hn = jnp.dot(x, w2_ref[...], preferred_element_type=_f32) + b2_ref[...]
    mu = jnp.mean(hn, axis=1, keepdims=True)
    var = jnp.mean((hn - mu) ** 2, axis=1, keepdims=True)
    hn = (hn - mu) / jnp.sqrt(var + 1e-5) * g_ref[...] + b_ref[...]
    out = h + hn
    olo_ref[...] = out[:, :HH]
    ohi_ref[...] = out[:, HH:]


def _update_call(h_lo, h_hi, agg_lo, agg_hi, cnt16, blk):
    grid = N // _TN
    half = pl.BlockSpec((_TN, HH), lambda i: (i, 0))
    full = lambda shape: pl.BlockSpec(shape, lambda i: (0, 0))
    return pl.pallas_call(
        _update_body,
        grid=(grid,),
        in_specs=[
            half, half, half, half,
            pl.BlockSpec((_TN, 16), lambda i: (i, 0)),
            full((2 * H, H)), full((1, H)),
            full((H, H)), full((1, H)),
            full((1, H)), full((1, H)),
        ],
        out_specs=[half, half],
        out_shape=[
            jax.ShapeDtypeStruct((N, HH), _f32),
            jax.ShapeDtypeStruct((N, HH), _f32),
        ],
    )(h_lo, h_hi, agg_lo, agg_hi, cnt16,
      blk['uw1'], blk['ub1'].reshape(1, H),
      blk['uw2'], blk['ub2'].reshape(1, H),
      blk['ln_g'].reshape(1, H), blk['ln_b'].reshape(1, H))


def _pool_body(b_ref, hlo_ref, hhi_ref,
               ew1_ref, eb1_ref, ew2_ref, eb2_ref,
               sw1_ref, sb1_ref, sw2_ref, sb2_ref,
               en_ref, sel_ref, acc_ref, cnt_ref):
    i = pl.program_id(0)

    @pl.when(i == 0)
    def _init():
        acc_ref[...] = jnp.zeros_like(acc_ref)
        cnt_ref[...] = jnp.zeros_like(cnt_ref)

    b = b_ref[...]
    oh = (b == lax.broadcasted_iota(_i32, (1, NG), 1)).astype(_f32)
    h = jnp.concatenate([hlo_ref[...], hhi_ref[...]], axis=1)
    dn = (((0,), (0,)), ((), ()))
    acc_ref[...] += lax.dot_general(oh, h, dn, preferred_element_type=_f32)
    cnt_ref[...] += lax.dot_general(oh, jnp.ones((oh.shape[0], 1), _f32), dn,
                                    preferred_element_type=_f32)

    @pl.when(i == pl.num_programs(0) - 1)
    def _final():
        pooled = acc_ref[...] / jnp.maximum(cnt_ref[...], 1.0)
        xe = jnp.maximum(jnp.dot(pooled, ew1_ref[...],
                                 preferred_element_type=_f32) + eb1_ref[...], 0.0)
        en_ref[...] = jnp.dot(xe, ew2_ref[...],
                              preferred_element_type=_f32) + eb2_ref[...]
        xs = jnp.maximum(jnp.dot(pooled, sw1_ref[...],
                                 preferred_element_type=_f32) + sb1_ref[...], 0.0)
        logits = jnp.dot(xs, sw2_ref[...],
                         preferred_element_type=_f32) + sb2_ref[...]
        z = logits - jnp.max(logits, axis=1, keepdims=True)
        ez = jnp.exp(z)
        sel_ref[...] = ez / jnp.sum(ez, axis=1, keepdims=True)


def _pool_call(b2, h_lo, h_hi, params):
    grid = N // _TN
    full = lambda shape: pl.BlockSpec(shape, lambda i: (0, 0))
    return pl.pallas_call(
        _pool_body,
        grid=(grid,),
        in_specs=[
            pl.BlockSpec((_TN, 1), lambda i: (i, 0)),
            pl.BlockSpec((_TN, HH), lambda i: (i, 0)),
            pl.BlockSpec((_TN, HH), lambda i: (i, 0)),
            full((H, H // 2)), full((1, H // 2)),
            full((H // 2, 1)), full((1, 1)),
            full((H, H // 2)), full((1, H // 2)),
            full((H // 2, 4)), full((1, 4)),
        ],
        out_specs=[full((NG, 1)), full((NG, 4))],
        out_shape=[
            jax.ShapeDtypeStruct((NG, 1), _f32),
            jax.ShapeDtypeStruct((NG, 4), _f32),
        ],
        scratch_shapes=[
            pltpu.VMEM((NG, H), _f32),
            pltpu.VMEM((NG, 1), _f32),
        ],
    )(b2, h_lo, h_hi,
      params['ew1'], params['eb1'].reshape(1, H // 2),
      params['ew2'], params['eb2'].reshape(1, 1),
      params['sw1'], params['sb1'].reshape(1, H // 2),
      params['sw2'], params['sb2'].reshape(1, 4))


@functools.lru_cache(maxsize=1)
def _sc_mesh():
    return plsc.VectorSubcoreMesh(core_axis_name="c", subcore_axis_name="s",
                                  num_cores=2, num_subcores=NSUB)


def _zero_rows(zbuf, width):
    def fill(r, _):
        zbuf[r, 0:16] = jnp.zeros((16,), _f32)
        if width > 16:
            zbuf[r, 16:32] = jnp.zeros((16,), _f32)
        return 0
    lax.fori_loop(0, ZROWS, fill, 0)


def _nchunks(s):
    return CH_PER_SUB + (s < CH_EXTRA).astype(_i32)


def _cnt_body(dst_hbm, out_hbm, idx_buf, ones_buf, zbuf, cnt_sh):
    c = lax.axis_index("c")
    s = lax.axis_index("s")

    @pl.when(c == 0)
    def _run():
        _zero_rows(zbuf, 16)

        def fill(r, _):
            ones_buf[r, 0:16] = jnp.ones((16,), _f32)
            return 0
        lax.fori_loop(0, CH, fill, 0)

        def zcp(k, _):
            pltpu.sync_copy(zbuf,
                            cnt_sh.at[pl.ds(s * ROWS_PER_SUB + k * ZROWS, ZROWS)])
            return 0
        lax.fori_loop(0, ROWS_PER_SUB // ZROWS, zcp, 0)
        plsc.subcore_barrier()

        def chunk(k, _):
            base = (s + k * NSUB) * CH
            pltpu.sync_copy(dst_hbm.at[pl.ds(base, CH)], idx_buf.at[0])
            pltpu.sync_copy(ones_buf, cnt_sh.at[idx_buf.at[0]], add=True)
            return 0
        lax.fori_loop(0, _nchunks(s), chunk, 0)
        plsc.subcore_barrier()

        pltpu.sync_copy(cnt_sh.at[pl.ds(s * ROWS_PER_SUB, ROWS_PER_SUB)],
                        out_hbm.at[pl.ds(s * ROWS_PER_SUB, ROWS_PER_SUB)])


def _cnt_call(dst):
    return pl.kernel(
        _cnt_body,
        out_type=[jax.ShapeDtypeStruct((NPAD, 16), _f32)],
        mesh=_sc_mesh(),
        scratch_types=[
            pltpu.VMEM((1, CH), _i32),
            pltpu.VMEM((CH, 16), _f32),
            pltpu.VMEM((ZROWS, 16), _f32),
            pltpu.VMEM_SHARED((NPAD, 16), _f32),
        ],
        compiler_params=pltpu.CompilerParams(use_tc_tiling_on_sc=False),
    )(dst)[0]


def _msg_half(h_hbm, w_hbm, agg_hbm, src_hbm, dst_hbm,
              idxs_buf, idxd_buf, rows_buf, w_buf, zbuf, agg_sh, s):
    _zero_rows(zbuf, HH)

    def zcp(k, _):
        pltpu.sync_copy(zbuf,
                        agg_sh.at[pl.ds(s * ROWS_PER_SUB + k * ZROWS, ZROWS)])
        return 0
    lax.fori_loop(0, ROWS_PER_SUB // ZROWS, zcp, 0)
    plsc.subcore_barrier()

    def chunk(k, _):
        base = (s + k * NSUB) * CH
        pltpu.sync_copy(src_hbm.at[pl.ds(base, CH)], idxs_buf.at[0])
        pltpu.sync_copy(dst_hbm.at[pl.ds(base, CH)], idxd_buf.at[0])
        pltpu.sync_copy(h_hbm.at[idxs_buf.at[0]], rows_buf)
        pltpu.sync_copy(w_hbm.at[pl.ds(base, CH)], w_buf)

        def mrow(r, _):
            rows_buf[r, 0:16] = rows_buf[r, 0:16] * w_buf[r, 0:16]
            rows_buf[r, 16:32] = rows_buf[r, 16:32] * w_buf[r, 16:32]
            return 0
        lax.fori_loop(0, CH, mrow, 0)

        pltpu.sync_copy(rows_buf, agg_sh.at[idxd_buf.at[0]], add=True)
        return 0
    lax.fori_loop(0, _nchunks(s), chunk, 0)
    plsc.subcore_barrier()

    pltpu.sync_copy(agg_sh.at[pl.ds(s * ROWS_PER_SUB, ROWS_PER_SUB)],
                    agg_hbm.at[pl.ds(s * ROWS_PER_SUB, ROWS_PER_SUB)])


def _msg_body(hlo_hbm, hhi_hbm, wlo_hbm, whi_hbm, src_hbm, dst_hbm,
              alo_hbm, ahi_hbm,
              idxs_buf, idxd_buf, rows_buf, w_buf, zbuf, agg_sh):
    c = lax.axis_index("c")
    s = lax.axis_index("s")

    @pl.when(c == 0)
    def _lo():
        _msg_half(hlo_hbm, wlo_hbm, alo_hbm, src_hbm, dst_hbm,
                  idxs_buf, idxd_buf, rows_buf, w_buf, zbuf, agg_sh, s)

    @pl.when(c == 1)
    def _hi():
        _msg_half(hhi_hbm, whi_hbm, ahi_hbm, src_hbm, dst_hbm,
                  idxs_buf, idxd_buf, rows_buf, w_buf, zbuf, agg_sh, s)


def _msg_call(h_lo, h_hi, w_lo, w_hi, src, dst):
    return pl.kernel(
        _msg_body,
        out_type=[
            jax.ShapeDtypeStruct((NPAD, HH), _f32),
            jax.ShapeDtypeStruct((NPAD, HH), _f32),
        ],
        mesh=_sc_mesh(),
        scratch_types=[
            pltpu.VMEM((1, CH), _i32),
            pltpu.VMEM((1, CH), _i32),
            pltpu.VMEM((CH, HH), _f32),
            pltpu.VMEM((CH, HH), _f32),
            pltpu.VMEM((ZROWS, HH), _f32),
            pltpu.VMEM_SHARED((NPAD, HH), _f32),
        ],
        compiler_params=pltpu.CompilerParams(use_tc_tiling_on_sc=False),
    )(h_lo, h_hi, w_lo, w_hi, src, dst)


def kernel(node_features, edge_index, edge_dist, batch, atomic_numbers, params):
    src = edge_index[0]
    dst = edge_index[1]
    an2 = atomic_numbers.reshape(N, 1).astype(_i32)
    b2 = batch.reshape(N, 1).astype(_i32)
    emb_pad = jnp.pad(params['embed'], ((0, 128 - params['embed'].shape[0]),
                                        (0, 0)))

    h_lo, h_hi = _embed_call(an2, node_features, emb_pad)
    cnt16 = _cnt_call(dst)

    for blk in params['blocks']:
        w_lo, w_hi = _filter_call(edge_dist, blk)
        agg_lo, agg_hi = _msg_call(h_lo, h_hi, w_lo, w_hi, src, dst)
        h_lo, h_hi = _update_call(h_lo, h_hi, agg_lo, agg_hi, cnt16, blk)

    energy, selectivity = _pool_call(b2, h_lo, h_hi, params)
    h = jnp.concatenate([h_lo, h_hi], axis=1)
    return energy, selectivity, h


# speedup vs baseline: 1.4350x; 1.4350x over previous
import functools
import math

import jax
import jax.numpy as jnp
from jax import lax
from jax.experimental import pallas as pl
from jax.experimental.pallas import tpu as pltpu
from jax.experimental.pallas import tpu_sc as plsc

N = 50000
E = 800000
H = 64
HH = 32
NRBF = 64
NF = 128
NG = 64
CUTOFF = 6.0
GAMMA = 10.0

NSUB = 16
CH = 128
NCHUNKS = E // CH
CH_PER_SUB = NCHUNKS // NSUB
CH_EXTRA = NCHUNKS - CH_PER_SUB * NSUB
NPAD = 50176
ROWS_PER_SUB = NPAD // NSUB
ZROWS = 392

_f32 = jnp.float32
_i32 = jnp.int32


_TN = 5000
_TE = 6400


def _embed_body(an_ref, nf_ref, emb_ref, lo_ref, hi_ref):
    a = an_ref[...]
    oh = (a == lax.broadcasted_iota(_i32, (1, 128), 1)).astype(_f32)
    h = jnp.dot(oh, emb_ref[...], preferred_element_type=_f32) + nf_ref[...]
    lo_ref[...] = h[:, :HH]
    hi_ref[...] = h[:, HH:]


def _embed_call(an2, node_features, emb_pad):
    grid = N // _TN
    return pl.pallas_call(
        _embed_body,
        grid=(grid,),
        in_specs=[
            pl.BlockSpec((_TN, 1), lambda i: (i, 0)),
            pl.BlockSpec((_TN, H), lambda i: (i, 0)),
            pl.BlockSpec((128, H), lambda i: (0, 0)),
        ],
        out_specs=[
            pl.BlockSpec((_TN, HH), lambda i: (i, 0)),
            pl.BlockSpec((_TN, HH), lambda i: (i, 0)),
        ],
        out_shape=[
            jax.ShapeDtypeStruct((N, HH), _f32),
            jax.ShapeDtypeStruct((N, HH), _f32),
        ],
    )(an2, node_features, emb_pad)


def _filter_body(d_ref, w1_ref, b1_ref, w2_ref, b2_ref, w3_ref, b3_ref,
                 lo_ref, hi_ref):
    d = d_ref[...]
    centers = lax.broadcasted_iota(_i32, (1, NRBF), 1).astype(_f32) * (
        CUTOFF / (NRBF - 1))
    rbf = jnp.exp(-GAMMA * (d - centers) ** 2)
    u = d * (math.pi / CUTOFF)
    t = u * u
    cosu = 1.0 + t * (-1.0 / 2 + t * (1.0 / 24 + t * (-1.0 / 720 + t * (
        1.0 / 40320 + t * (-1.0 / 3628800 + t * (1.0 / 479001600))))))
    cf = 0.5 * (cosu + 1.0)
    cf = cf * (d < CUTOFF).astype(_f32)
    rbf = rbf * cf
    bf16 = jnp.bfloat16
    x = jnp.maximum(jnp.dot(rbf.astype(bf16), w1_ref[...].astype(bf16),
                            preferred_element_type=_f32) + b1_ref[...], 0.0)
    x = jnp.maximum(jnp.dot(x.astype(bf16), w2_ref[...].astype(bf16),
                            preferred_element_type=_f32) + b2_ref[...], 0.0)
    w = jnp.dot(x.astype(bf16), w3_ref[...].astype(bf16),
                preferred_element_type=_f32) + b3_ref[...]
    lo_ref[...] = w[:, :HH]
    hi_ref[...] = w[:, HH:]


def _filter_call(d, blk):
    grid = E // _TE
    full = lambda shape: pl.BlockSpec(shape, lambda i: (0, 0))
    return pl.pallas_call(
        _filter_body,
        grid=(grid,),
        in_specs=[
            pl.BlockSpec((_TE, 1), lambda i: (i, 0)),
            full((NRBF, NF)), full((1, NF)),
            full((NF, NF)), full((1, NF)),
            full((NF, H)), full((1, H)),
        ],
        out_specs=[
            pl.BlockSpec((_TE, HH), lambda i: (i, 0)),
            pl.BlockSpec((_TE, HH), lambda i: (i, 0)),
        ],
        out_shape=[
            jax.ShapeDtypeStruct((E, HH), _f32),
            jax.ShapeDtypeStruct((E, HH), _f32),
        ],
    )(d, blk['fw1'], blk['fb1'].reshape(1, NF),
      blk['fw2'], blk['fb2'].reshape(1, NF),
      blk['fw3'], blk['fb3'].reshape(1, H))


def _update_body(hlo_ref, hhi_ref, alo_ref, ahi_ref, cnt_ref,
                 w1_ref, b1_ref, w2_ref, b2_ref, g_ref, b_ref,
                 olo_ref, ohi_ref):
    h = jnp.concatenate([hlo_ref[...], hhi_ref[...]], axis=1)
    rc = 1.0 / jnp.maximum(cnt_ref[...][:, :1], 1.0)
    agg = jnp.concatenate([alo_ref[...], ahi_ref[...]], axis=1) * rc
    comb = jnp.concatenate([h, agg], axis=1)
    x = jnp.maximum(jnp.dot(comb, w1_ref[...], preferred_element_type=_f32)
                    + b1_ref[...], 0.0)
    hn = jnp.dot(x, w2_ref[...], preferred_element_type=_f32) + b2_ref[...]
    mu = jnp.mean(hn, axis=1, keepdims=True)
    var = jnp.mean((hn - mu) ** 2, axis=1, keepdims=True)
    hn = (hn - mu) / jnp.sqrt(var + 1e-5) * g_ref[...] + b_ref[...]
    out = h + hn
    olo_ref[...] = out[:, :HH]
    ohi_ref[...] = out[:, HH:]


def _update_call(h_lo, h_hi, agg_lo, agg_hi, cnt16, blk):
    grid = N // _TN
    half = pl.BlockSpec((_TN, HH), lambda i: (i, 0))
    full = lambda shape: pl.BlockSpec(shape, lambda i: (0, 0))
    return pl.pallas_call(
        _update_body,
        grid=(grid,),
        in_specs=[
            half, half, half, half,
            pl.BlockSpec((_TN, 16), lambda i: (i, 0)),
            full((2 * H, H)), full((1, H)),
            full((H, H)), full((1, H)),
            full((1, H)), full((1, H)),
        ],
        out_specs=[half, half],
        out_shape=[
            jax.ShapeDtypeStruct((N, HH), _f32),
            jax.ShapeDtypeStruct((N, HH), _f32),
        ],
    )(h_lo, h_hi, agg_lo, agg_hi, cnt16,
      blk['uw1'], blk['ub1'].reshape(1, H),
      blk['uw2'], blk['ub2'].reshape(1, H),
      blk['ln_g'].reshape(1, H), blk['ln_b'].reshape(1, H))


def _pool_body(b_ref, hlo_ref, hhi_ref,
               ew1_ref, eb1_ref, ew2_ref, eb2_ref,
               sw1_ref, sb1_ref, sw2_ref, sb2_ref,
               en_ref, sel_ref, acc_ref, cnt_ref):
    i = pl.program_id(0)

    @pl.when(i == 0)
    def _init():
        acc_ref[...] = jnp.zeros_like(acc_ref)
        cnt_ref[...] = jnp.zeros_like(cnt_ref)

    b = b_ref[...]
    oh = (b == lax.broadcasted_iota(_i32, (1, NG), 1)).astype(_f32)
    h = jnp.concatenate([hlo_ref[...], hhi_ref[...]], axis=1)
    dn = (((0,), (0,)), ((), ()))
    acc_ref[...] += lax.dot_general(oh, h, dn, preferred_element_type=_f32)
    cnt_ref[...] += lax.dot_general(oh, jnp.ones((oh.shape[0], 1), _f32), dn,
                                    preferred_element_type=_f32)

    @pl.when(i == pl.num_programs(0) - 1)
    def _final():
        pooled = acc_ref[...] / jnp.maximum(cnt_ref[...], 1.0)
        xe = jnp.maximum(jnp.dot(pooled, ew1_ref[...],
                                 preferred_element_type=_f32) + eb1_ref[...], 0.0)
        en_ref[...] = jnp.dot(xe, ew2_ref[...],
                              preferred_element_type=_f32) + eb2_ref[...]
        xs = jnp.maximum(jnp.dot(pooled, sw1_ref[...],
                                 preferred_element_type=_f32) + sb1_ref[...], 0.0)
        logits = jnp.dot(xs, sw2_ref[...],
                         preferred_element_type=_f32) + sb2_ref[...]
        z = logits - jnp.max(logits, axis=1, keepdims=True)
        ez = jnp.exp(z)
        sel_ref[...] = ez / jnp.sum(ez, axis=1, keepdims=True)


def _pool_call(b2, h_lo, h_hi, params):
    grid = N // _TN
    full = lambda shape: pl.BlockSpec(shape, lambda i: (0, 0))
    return pl.pallas_call(
        _pool_body,
        grid=(grid,),
        in_specs=[
            pl.BlockSpec((_TN, 1), lambda i: (i, 0)),
            pl.BlockSpec((_TN, HH), lambda i: (i, 0)),
            pl.BlockSpec((_TN, HH), lambda i: (i, 0)),
            full((H, H // 2)), full((1, H // 2)),
            full((H // 2, 1)), full((1, 1)),
            full((H, H // 2)), full((1, H // 2)),
            full((H // 2, 4)), full((1, 4)),
        ],
        out_specs=[full((NG, 1)), full((NG, 4))],
        out_shape=[
            jax.ShapeDtypeStruct((NG, 1), _f32),
            jax.ShapeDtypeStruct((NG, 4), _f32),
        ],
        scratch_shapes=[
            pltpu.VMEM((NG, H), _f32),
            pltpu.VMEM((NG, 1), _f32),
        ],
    )(b2, h_lo, h_hi,
      params['ew1'], params['eb1'].reshape(1, H // 2),
      params['ew2'], params['eb2'].reshape(1, 1),
      params['sw1'], params['sb1'].reshape(1, H // 2),
      params['sw2'], params['sb2'].reshape(1, 4))


@functools.lru_cache(maxsize=1)
def _sc_mesh():
    return plsc.VectorSubcoreMesh(core_axis_name="c", subcore_axis_name="s",
                                  num_cores=2, num_subcores=NSUB)


def _zero_rows(zbuf, width):
    def fill(r, _):
        zbuf[r, 0:16] = jnp.zeros((16,), _f32)
        if width > 16:
            zbuf[r, 16:32] = jnp.zeros((16,), _f32)
        return 0
    lax.fori_loop(0, ZROWS, fill, 0)


def _nchunks(s):
    return CH_PER_SUB + (s < CH_EXTRA).astype(_i32)


def _cnt_body(dst_hbm, out_hbm, idx_buf, ones_buf, zbuf, cnt_sh):
    c = lax.axis_index("c")
    s = lax.axis_index("s")

    @pl.when(c == 0)
    def _run():
        _zero_rows(zbuf, 16)

        def fill(r, _):
            ones_buf[r, 0:16] = jnp.ones((16,), _f32)
            return 0
        lax.fori_loop(0, CH, fill, 0)

        def zcp(k, _):
            pltpu.sync_copy(zbuf,
                            cnt_sh.at[pl.ds(s * ROWS_PER_SUB + k * ZROWS, ZROWS)])
            return 0
        lax.fori_loop(0, ROWS_PER_SUB // ZROWS, zcp, 0)
        plsc.subcore_barrier()

        def chunk(k, _):
            base = (s + k * NSUB) * CH
            pltpu.sync_copy(dst_hbm.at[pl.ds(base, CH)], idx_buf.at[0])
            pltpu.sync_copy(ones_buf, cnt_sh.at[idx_buf.at[0]], add=True)
            return 0
        lax.fori_loop(0, _nchunks(s), chunk, 0)
        plsc.subcore_barrier()

        pltpu.sync_copy(cnt_sh.at[pl.ds(s * ROWS_PER_SUB, ROWS_PER_SUB)],
                        out_hbm.at[pl.ds(s * ROWS_PER_SUB, ROWS_PER_SUB)])


def _cnt_call(dst):
    return pl.kernel(
        _cnt_body,
        out_type=[jax.ShapeDtypeStruct((NPAD, 16), _f32)],
        mesh=_sc_mesh(),
        scratch_types=[
            pltpu.VMEM((1, CH), _i32),
            pltpu.VMEM((CH, 16), _f32),
            pltpu.VMEM((ZROWS, 16), _f32),
            pltpu.VMEM_SHARED((NPAD, 16), _f32),
        ],
        compiler_params=pltpu.CompilerParams(use_tc_tiling_on_sc=False),
    )(dst)[0]


def _msg_half(h_hbm, w_hbm, agg_hbm, src_hbm, dst_hbm,
              idxs_buf, idxd_buf, rows_buf, w_buf, zbuf,
              gsem0, gsem1, wsem0, wsem1, agg_sh, s):
    _zero_rows(zbuf, HH)

    def zcp(k, _):
        pltpu.sync_copy(zbuf,
                        agg_sh.at[pl.ds(s * ROWS_PER_SUB + k * ZROWS, ZROWS)])
        return 0
    lax.fori_loop(0, ROWS_PER_SUB // ZROWS, zcp, 0)
    plsc.subcore_barrier()

    bufs = ((idxs_buf[0], idxd_buf[0], rows_buf[0], w_buf[0], gsem0, wsem0),
            (idxs_buf[1], idxd_buf[1], rows_buf[1], w_buf[1], gsem1, wsem1))

    def load(k, b):
        isrc, idst, rows, wv, gsem, wsem = bufs[b]
        base = (s + k * NSUB) * CH
        pltpu.sync_copy(src_hbm.at[pl.ds(base, CH)], isrc.at[0])
        pltpu.sync_copy(dst_hbm.at[pl.ds(base, CH)], idst.at[0])
        pltpu.sync_copy(h_hbm.at[isrc.at[0]], rows)
        pltpu.sync_copy(w_hbm.at[pl.ds(base, CH)], wv)
        return None, None

    def proc(b, g, w):
        isrc, idst, rows, wv, gsem, wsem = bufs[b]

        def mrow(q, _):
            for j in range(4):
                r = q * 4 + j
                rows[r, 0:16] = rows[r, 0:16] * wv[r, 0:16]
                rows[r, 16:32] = rows[r, 16:32] * wv[r, 16:32]
            return 0
        lax.fori_loop(0, CH // 4, mrow, 0)
        pltpu.sync_copy(rows, agg_sh.at[idst.at[0]], add=True)

    nc = _nchunks(s)

    def pair(p, _):
        g0, w0 = load(2 * p, 0)
        g1, w1 = load(2 * p + 1, 1)
        proc(0, g0, w0)
        proc(1, g1, w1)
        return 0
    lax.fori_loop(0, nc // 2, pair, 0)

    @pl.when(nc % 2 == 1)
    def _rem():
        g, w = load(nc - 1, 0)
        proc(0, g, w)

    plsc.subcore_barrier()

    pltpu.sync_copy(agg_sh.at[pl.ds(s * ROWS_PER_SUB, ROWS_PER_SUB)],
                    agg_hbm.at[pl.ds(s * ROWS_PER_SUB, ROWS_PER_SUB)])


def _msg_body(hlo_hbm, hhi_hbm, wlo_hbm, whi_hbm, src_hbm, dst_hbm,
              alo_hbm, ahi_hbm,
              idxs_buf, idxd_buf, rows_buf, w_buf, zbuf,
              gsem0, gsem1, wsem0, wsem1, agg_sh):
    c = lax.axis_index("c")
    s = lax.axis_index("s")

    @pl.when(c == 0)
    def _lo():
        _msg_half(hlo_hbm, wlo_hbm, alo_hbm, src_hbm, dst_hbm,
                  idxs_buf, idxd_buf, rows_buf, w_buf, zbuf,
                  gsem0, gsem1, wsem0, wsem1, agg_sh, s)

    @pl.when(c == 1)
    def _hi():
        _msg_half(hhi_hbm, whi_hbm, ahi_hbm, src_hbm, dst_hbm,
                  idxs_buf, idxd_buf, rows_buf, w_buf, zbuf,
                  gsem0, gsem1, wsem0, wsem1, agg_sh, s)


def _msg_call(h_lo, h_hi, w_lo, w_hi, src, dst):
    return pl.kernel(
        _msg_body,
        out_type=[
            jax.ShapeDtypeStruct((NPAD, HH), _f32),
            jax.ShapeDtypeStruct((NPAD, HH), _f32),
        ],
        mesh=_sc_mesh(),
        scratch_types=[
            [pltpu.VMEM((1, CH), _i32), pltpu.VMEM((1, CH), _i32)],
            [pltpu.VMEM((1, CH), _i32), pltpu.VMEM((1, CH), _i32)],
            [pltpu.VMEM((CH, HH), _f32), pltpu.VMEM((CH, HH), _f32)],
            [pltpu.VMEM((CH, HH), _f32), pltpu.VMEM((CH, HH), _f32)],
            pltpu.VMEM((ZROWS, HH), _f32),
            pltpu.SemaphoreType.DMA,
            pltpu.SemaphoreType.DMA,
            pltpu.SemaphoreType.DMA,
            pltpu.SemaphoreType.DMA,
            pltpu.VMEM_SHARED((NPAD, HH), _f32),
        ],
        compiler_params=pltpu.CompilerParams(use_tc_tiling_on_sc=False),
    )(h_lo, h_hi, w_lo, w_hi, src, dst)


def kernel(node_features, edge_index, edge_dist, batch, atomic_numbers, params):
    src = edge_index[0]
    dst = edge_index[1]
    an2 = atomic_numbers.reshape(N, 1).astype(_i32)
    b2 = batch.reshape(N, 1).astype(_i32)
    emb_pad = jnp.pad(params['embed'], ((0, 128 - params['embed'].shape[0]),
                                        (0, 0)))

    h_lo, h_hi = _embed_call(an2, node_features, emb_pad)
    cnt16 = _cnt_call(dst)

    for blk in params['blocks']:
        w_lo, w_hi = _filter_call(edge_dist, blk)
        agg_lo, agg_hi = _msg_call(h_lo, h_hi, w_lo, w_hi, src, dst)
        h_lo, h_hi = _update_call(h_lo, h_hi, agg_lo, agg_hi, cnt16, blk)

    energy, selectivity = _pool_call(b2, h_lo, h_hi, params)
    h = jnp.concatenate([h_lo, h_hi], axis=1)
    return energy, selectivity, h


# speedup vs baseline: 1.9750x; 1.3763x over previous
import functools
import math

import jax
import jax.numpy as jnp
from jax import lax
from jax.experimental import pallas as pl
from jax.experimental.pallas import tpu as pltpu
from jax.experimental.pallas import tpu_sc as plsc

N = 50000
E = 800000
H = 64
HH = 32
NRBF = 64
NF = 128
NG = 64
CUTOFF = 6.0
GAMMA = 10.0

NSUB = 16
CH = 128
NCHUNKS = E // CH
CH_PER_SUB = NCHUNKS // NSUB
CH_EXTRA = NCHUNKS - CH_PER_SUB * NSUB
CPS = 392
SS = 3
ZCP = 196
NPAD = 50176
ROWS_PER_SUB = NPAD // NSUB
ZROWS = 392

_f32 = jnp.float32
_i32 = jnp.int32


_TN = 5000
_TE = 6400


def _embed_body(an_ref, nf_ref, emb_ref, lo_ref, hi_ref):
    a = an_ref[...]
    oh = (a == lax.broadcasted_iota(_i32, (1, 128), 1)).astype(_f32)
    h = jnp.dot(oh, emb_ref[...], preferred_element_type=_f32) + nf_ref[...]
    lo_ref[...] = h[:, :HH]
    hi_ref[...] = h[:, HH:]


def _embed_call(an2, node_features, emb_pad):
    grid = N // _TN
    return pl.pallas_call(
        _embed_body,
        grid=(grid,),
        in_specs=[
            pl.BlockSpec((_TN, 1), lambda i: (i, 0)),
            pl.BlockSpec((_TN, H), lambda i: (i, 0)),
            pl.BlockSpec((128, H), lambda i: (0, 0)),
        ],
        out_specs=[
            pl.BlockSpec((_TN, HH), lambda i: (i, 0)),
            pl.BlockSpec((_TN, HH), lambda i: (i, 0)),
        ],
        out_shape=[
            jax.ShapeDtypeStruct((N, HH), _f32),
            jax.ShapeDtypeStruct((N, HH), _f32),
        ],
    )(an2, node_features, emb_pad)


def _filter_body(d_ref, w1_ref, b1_ref, w2_ref, b2_ref, w3_ref, b3_ref,
                 lo_ref, hi_ref):
    d = d_ref[...]
    centers = lax.broadcasted_iota(_i32, (1, NRBF), 1).astype(_f32) * (
        CUTOFF / (NRBF - 1))
    rbf = jnp.exp(-GAMMA * (d - centers) ** 2)
    u = d * (math.pi / CUTOFF)
    t = u * u
    cosu = 1.0 + t * (-1.0 / 2 + t * (1.0 / 24 + t * (-1.0 / 720 + t * (
        1.0 / 40320 + t * (-1.0 / 3628800 + t * (1.0 / 479001600))))))
    cf = 0.5 * (cosu + 1.0)
    cf = cf * (d < CUTOFF).astype(_f32)
    rbf = rbf * cf
    bf16 = jnp.bfloat16
    x = jnp.maximum(jnp.dot(rbf.astype(bf16), w1_ref[...].astype(bf16),
                            preferred_element_type=_f32) + b1_ref[...], 0.0)
    x = jnp.maximum(jnp.dot(x.astype(bf16), w2_ref[...].astype(bf16),
                            preferred_element_type=_f32) + b2_ref[...], 0.0)
    w = jnp.dot(x.astype(bf16), w3_ref[...].astype(bf16),
                preferred_element_type=_f32) + b3_ref[...]
    lo_ref[...] = w[:, :HH]
    hi_ref[...] = w[:, HH:]


def _filter_call(d, blk):
    grid = E // _TE
    full = lambda shape: pl.BlockSpec(shape, lambda i: (0, 0))
    return pl.pallas_call(
        _filter_body,
        grid=(grid,),
        in_specs=[
            pl.BlockSpec((_TE, 1), lambda i: (i, 0)),
            full((NRBF, NF)), full((1, NF)),
            full((NF, NF)), full((1, NF)),
            full((NF, H)), full((1, H)),
        ],
        out_specs=[
            pl.BlockSpec((_TE, HH), lambda i: (i, 0)),
            pl.BlockSpec((_TE, HH), lambda i: (i, 0)),
        ],
        out_shape=[
            jax.ShapeDtypeStruct((E, HH), _f32),
            jax.ShapeDtypeStruct((E, HH), _f32),
        ],
    )(d, blk['fw1'], blk['fb1'].reshape(1, NF),
      blk['fw2'], blk['fb2'].reshape(1, NF),
      blk['fw3'], blk['fb3'].reshape(1, H))


def _update_body(hlo_ref, hhi_ref, alo_ref, ahi_ref, cnt_ref,
                 w1_ref, b1_ref, w2_ref, b2_ref, g_ref, b_ref,
                 olo_ref, ohi_ref):
    h = jnp.concatenate([hlo_ref[...], hhi_ref[...]], axis=1)
    rc = 1.0 / jnp.maximum(cnt_ref[...][:, :1], 1.0)
    agg = jnp.concatenate([alo_ref[...], ahi_ref[...]], axis=1) * rc
    comb = jnp.concatenate([h, agg], axis=1)
    x = jnp.maximum(jnp.dot(comb, w1_ref[...], preferred_element_type=_f32)
                    + b1_ref[...], 0.0)
    hn = jnp.dot(x, w2_ref[...], preferred_element_type=_f32) + b2_ref[...]
    mu = jnp.mean(hn, axis=1, keepdims=True)
    var = jnp.mean((hn - mu) ** 2, axis=1, keepdims=True)
    hn = (hn - mu) / jnp.sqrt(var + 1e-5) * g_ref[...] + b_ref[...]
    out = h + hn
    olo_ref[...] = out[:, :HH]
    ohi_ref[...] = out[:, HH:]


def _update_call(h_lo, h_hi, agg_lo, agg_hi, cnt16, blk):
    grid = N // _TN
    half = pl.BlockSpec((_TN, HH), lambda i: (i, 0))
    full = lambda shape: pl.BlockSpec(shape, lambda i: (0, 0))
    return pl.pallas_call(
        _update_body,
        grid=(grid,),
        in_specs=[
            half, half, half, half,
            pl.BlockSpec((_TN, 16), lambda i: (i, 0)),
            full((2 * H, H)), full((1, H)),
            full((H, H)), full((1, H)),
            full((1, H)), full((1, H)),
        ],
        out_specs=[half, half],
        out_shape=[
            jax.ShapeDtypeStruct((N, HH), _f32),
            jax.ShapeDtypeStruct((N, HH), _f32),
        ],
    )(h_lo, h_hi, agg_lo, agg_hi, cnt16,
      blk['uw1'], blk['ub1'].reshape(1, H),
      blk['uw2'], blk['ub2'].reshape(1, H),
      blk['ln_g'].reshape(1, H), blk['ln_b'].reshape(1, H))


def _pool_body(b_ref, hlo_ref, hhi_ref,
               ew1_ref, eb1_ref, ew2_ref, eb2_ref,
               sw1_ref, sb1_ref, sw2_ref, sb2_ref,
               en_ref, sel_ref, acc_ref, cnt_ref):
    i = pl.program_id(0)

    @pl.when(i == 0)
    def _init():
        acc_ref[...] = jnp.zeros_like(acc_ref)
        cnt_ref[...] = jnp.zeros_like(cnt_ref)

    b = b_ref[...]
    oh = (b == lax.broadcasted_iota(_i32, (1, NG), 1)).astype(_f32)
    h = jnp.concatenate([hlo_ref[...], hhi_ref[...]], axis=1)
    dn = (((0,), (0,)), ((), ()))
    acc_ref[...] += lax.dot_general(oh, h, dn, preferred_element_type=_f32)
    cnt_ref[...] += lax.dot_general(oh, jnp.ones((oh.shape[0], 1), _f32), dn,
                                    preferred_element_type=_f32)

    @pl.when(i == pl.num_programs(0) - 1)
    def _final():
        pooled = acc_ref[...] / jnp.maximum(cnt_ref[...], 1.0)
        xe = jnp.maximum(jnp.dot(pooled, ew1_ref[...],
                                 preferred_element_type=_f32) + eb1_ref[...], 0.0)
        en_ref[...] = jnp.dot(xe, ew2_ref[...],
                              preferred_element_type=_f32) + eb2_ref[...]
        xs = jnp.maximum(jnp.dot(pooled, sw1_ref[...],
                                 preferred_element_type=_f32) + sb1_ref[...], 0.0)
        logits = jnp.dot(xs, sw2_ref[...],
                         preferred_element_type=_f32) + sb2_ref[...]
        z = logits - jnp.max(logits, axis=1, keepdims=True)
        ez = jnp.exp(z)
        sel_ref[...] = ez / jnp.sum(ez, axis=1, keepdims=True)


def _pool_call(b2, h_lo, h_hi, params):
    grid = N // _TN
    full = lambda shape: pl.BlockSpec(shape, lambda i: (0, 0))
    return pl.pallas_call(
        _pool_body,
        grid=(grid,),
        in_specs=[
            pl.BlockSpec((_TN, 1), lambda i: (i, 0)),
            pl.BlockSpec((_TN, HH), lambda i: (i, 0)),
            pl.BlockSpec((_TN, HH), lambda i: (i, 0)),
            full((H, H // 2)), full((1, H // 2)),
            full((H // 2, 1)), full((1, 1)),
            full((H, H // 2)), full((1, H // 2)),
            full((H // 2, 4)), full((1, 4)),
        ],
        out_specs=[full((NG, 1)), full((NG, 4))],
        out_shape=[
            jax.ShapeDtypeStruct((NG, 1), _f32),
            jax.ShapeDtypeStruct((NG, 4), _f32),
        ],
        scratch_shapes=[
            pltpu.VMEM((NG, H), _f32),
            pltpu.VMEM((NG, 1), _f32),
        ],
    )(b2, h_lo, h_hi,
      params['ew1'], params['eb1'].reshape(1, H // 2),
      params['ew2'], params['eb2'].reshape(1, 1),
      params['sw1'], params['sb1'].reshape(1, H // 2),
      params['sw2'], params['sb2'].reshape(1, 4))


@functools.lru_cache(maxsize=1)
def _sc_mesh():
    return plsc.VectorSubcoreMesh(core_axis_name="c", subcore_axis_name="s",
                                  num_cores=2, num_subcores=NSUB)


def _zero_rows(zbuf, width):
    def fill(r, _):
        zbuf[r, 0:16] = jnp.zeros((16,), _f32)
        if width > 16:
            zbuf[r, 16:32] = jnp.zeros((16,), _f32)
        return 0
    lax.fori_loop(0, ZROWS, fill, 0)


def _nchunks(s):
    return CH_PER_SUB + (s < CH_EXTRA).astype(_i32)


def _cnt_body(dst_hbm, out_hbm, idx_buf, ones_buf, zbuf, cnt_sh):
    c = lax.axis_index("c")
    s = lax.axis_index("s")

    @pl.when(c == 0)
    def _run():
        _zero_rows(zbuf, 16)

        def fill(r, _):
            ones_buf[r, 0:16] = jnp.ones((16,), _f32)
            return 0
        lax.fori_loop(0, CH, fill, 0)

        def zcp(k, _):
            pltpu.sync_copy(zbuf,
                            cnt_sh.at[pl.ds(s * ROWS_PER_SUB + k * ZROWS, ZROWS)])
            return 0
        lax.fori_loop(0, ROWS_PER_SUB // ZROWS, zcp, 0)
        plsc.subcore_barrier()

        def chunk(k, _):
            base = (s + k * NSUB) * CH
            pltpu.sync_copy(dst_hbm.at[pl.ds(base, CH)], idx_buf.at[0])
            pltpu.sync_copy(ones_buf, cnt_sh.at[idx_buf.at[0]], add=True)
            return 0
        lax.fori_loop(0, _nchunks(s), chunk, 0)
        plsc.subcore_barrier()

        pltpu.sync_copy(cnt_sh.at[pl.ds(s * ROWS_PER_SUB, ROWS_PER_SUB)],
                        out_hbm.at[pl.ds(s * ROWS_PER_SUB, ROWS_PER_SUB)])


def _cnt_call(dst):
    return pl.kernel(
        _cnt_body,
        out_type=[jax.ShapeDtypeStruct((NPAD, 16), _f32)],
        mesh=_sc_mesh(),
        scratch_types=[
            pltpu.VMEM((1, CH), _i32),
            pltpu.VMEM((CH, 16), _f32),
            pltpu.VMEM((ZROWS, 16), _f32),
            pltpu.VMEM_SHARED((NPAD, 16), _f32),
        ],
        compiler_params=pltpu.CompilerParams(use_tc_tiling_on_sc=False),
    )(dst)[0]


def _msg_half(h_hbm, w_hbm, agg_hbm, src1, dst1,
              isrc2, idxd2, rows, wbuf, gsem, wsem, ssem, isem,
              agg_sh, s):
    def zfill(r, _):
        rows[r, 0:16] = jnp.zeros((16,), _f32)
        rows[r, 16:32] = jnp.zeros((16,), _f32)
        return 0
    lax.fori_loop(0, ZCP, zfill, 0)

    def zcp(k, _):
        pltpu.sync_copy(rows.at[pl.ds(0, ZCP)],
                        agg_sh.at[pl.ds(s * ROWS_PER_SUB + k * ZCP, ZCP)])
        return 0
    lax.fori_loop(0, ROWS_PER_SUB // ZCP, zcp, 0)
    plsc.subcore_barrier()

    def mult(n4):
        def mrow(q, _):
            for j in range(4):
                r = q * 4 + j
                rows[r, 0:16] = rows[r, 0:16] * wbuf[r, 0:16]
                rows[r, 16:32] = rows[r, 16:32] * wbuf[r, 16:32]
            return 0
        lax.fori_loop(0, n4, mrow, 0)

    start_c = CPS * s
    nc = jnp.minimum(jnp.maximum(NCHUNKS - start_c, 0), CPS)
    nsc = nc // SS

    def superchunk(p, _):
        c0 = start_c + SS * p
        ids = [pltpu.async_copy(src1.at[pl.ds((c0 + j) * CH, CH)],
                                isrc2.at[j], isem) for j in range(SS)]
        ids += [pltpu.async_copy(dst1.at[pl.ds((c0 + j) * CH, CH)],
                                 idxd2.at[j], isem) for j in range(SS)]
        for d in ids:
            d.wait()
        gds = [pltpu.async_copy(h_hbm.at[isrc2.at[j]],
                                rows.at[pl.ds(j * CH, CH)], gsem)
               for j in range(SS)]
        wd = pltpu.async_copy(w_hbm.at[pl.ds(c0 * CH, SS * CH)], wbuf, wsem)
        for d in gds:
            d.wait()
        wd.wait()
        mult(SS * CH // 4)
        sds = [pltpu.async_copy(rows.at[pl.ds(j * CH, CH)],
                                agg_sh.at[idxd2.at[j]], ssem, add=True)
               for j in range(SS)]
        for d in sds:
            d.wait()
        return 0
    lax.fori_loop(0, nsc, superchunk, 0)

    def single(r, _):
        base = (start_c + nsc * SS + r) * CH
        pltpu.sync_copy(src1.at[pl.ds(base, CH)], isrc2.at[0])
        pltpu.sync_copy(dst1.at[pl.ds(base, CH)], idxd2.at[0])
        pltpu.sync_copy(h_hbm.at[isrc2.at[0]], rows.at[pl.ds(0, CH)])
        pltpu.sync_copy(w_hbm.at[pl.ds(base, CH)], wbuf.at[pl.ds(0, CH)])
        mult(CH // 4)
        pltpu.sync_copy(rows.at[pl.ds(0, CH)], agg_sh.at[idxd2.at[0]],
                        add=True)
        return 0
    lax.fori_loop(0, nc - nsc * SS, single, 0)

    plsc.subcore_barrier()

    pltpu.sync_copy(agg_sh.at[pl.ds(s * ROWS_PER_SUB, ROWS_PER_SUB)],
                    agg_hbm.at[pl.ds(s * ROWS_PER_SUB, ROWS_PER_SUB)])


def _msg_body(hlo_hbm, hhi_hbm, wlo_hbm, whi_hbm,
              src1_hbm, dst1_hbm,
              alo_hbm, ahi_hbm,
              isrc2, idxd2, rows, wbuf, gsem, wsem, ssem, isem,
              agg_sh):
    c = lax.axis_index("c")
    s = lax.axis_index("s")

    @pl.when(c == 0)
    def _lo():
        _msg_half(hlo_hbm, wlo_hbm, alo_hbm, src1_hbm, dst1_hbm,
                  isrc2, idxd2, rows, wbuf,
                  gsem, wsem, ssem, isem, agg_sh, s)

    @pl.when(c == 1)
    def _hi():
        _msg_half(hhi_hbm, whi_hbm, ahi_hbm, src1_hbm, dst1_hbm,
                  isrc2, idxd2, rows, wbuf,
                  gsem, wsem, ssem, isem, agg_sh, s)


def _msg_call(h_lo, h_hi, w_lo, w_hi, src, dst):
    return pl.kernel(
        _msg_body,
        out_type=[
            jax.ShapeDtypeStruct((NPAD, HH), _f32),
            jax.ShapeDtypeStruct((NPAD, HH), _f32),
        ],
        mesh=_sc_mesh(),
        scratch_types=[
            pltpu.VMEM((SS, CH), _i32),
            pltpu.VMEM((SS, CH), _i32),
            pltpu.VMEM((SS * CH, HH), _f32),
            pltpu.VMEM((SS * CH, HH), _f32),
            pltpu.SemaphoreType.DMA,
            pltpu.SemaphoreType.DMA,
            pltpu.SemaphoreType.DMA,
            pltpu.SemaphoreType.DMA,
            pltpu.VMEM_SHARED((NPAD, HH), _f32),
        ],
        compiler_params=pltpu.CompilerParams(use_tc_tiling_on_sc=False),
    )(h_lo, h_hi, w_lo, w_hi, src, dst)


def kernel(node_features, edge_index, edge_dist, batch, atomic_numbers, params):
    src = edge_index[0]
    dst = edge_index[1]
    an2 = atomic_numbers.reshape(N, 1).astype(_i32)
    b2 = batch.reshape(N, 1).astype(_i32)
    emb_pad = jnp.pad(params['embed'], ((0, 128 - params['embed'].shape[0]),
                                        (0, 0)))

    h_lo, h_hi = _embed_call(an2, node_features, emb_pad)
    cnt16 = _cnt_call(dst)

    for blk in params['blocks']:
        w_lo, w_hi = _filter_call(edge_dist, blk)
        agg_lo, agg_hi = _msg_call(h_lo, h_hi, w_lo, w_hi, src, dst)
        h_lo, h_hi = _update_call(h_lo, h_hi, agg_lo, agg_hi, cnt16, blk)

    energy, selectivity = _pool_call(b2, h_lo, h_hi, params)
    h = jnp.concatenate([h_lo, h_hi], axis=1)
    return energy, selectivity, h


# speedup vs baseline: 1.9760x; 1.0005x over previous
import functools
import math

import jax
import jax.numpy as jnp
from jax import lax
from jax.experimental import pallas as pl
from jax.experimental.pallas import tpu as pltpu
from jax.experimental.pallas import tpu_sc as plsc

N = 50000
E = 800000
H = 64
HH = 32
NRBF = 64
NF = 128
NG = 64
CUTOFF = 6.0
GAMMA = 10.0

NSUB = 16
CH = 128
NCHUNKS = E // CH
CH_PER_SUB = NCHUNKS // NSUB
CH_EXTRA = NCHUNKS - CH_PER_SUB * NSUB
CPS = 392
SS = 3
ZCP = 196
NPAD = 50176
ROWS_PER_SUB = NPAD // NSUB
ZROWS = 392

_f32 = jnp.float32
_i32 = jnp.int32


_TN = 5000
_TE = 6400


def _embed_body(an_ref, nf_ref, emb_ref, lo_ref, hi_ref):
    a = an_ref[...]
    oh = (a == lax.broadcasted_iota(_i32, (1, 128), 1)).astype(_f32)
    h = jnp.dot(oh, emb_ref[...], preferred_element_type=_f32) + nf_ref[...]
    lo_ref[...] = h[:, :HH]
    hi_ref[...] = h[:, HH:]


def _embed_call(an2, node_features, emb_pad):
    grid = N // _TN
    return pl.pallas_call(
        _embed_body,
        grid=(grid,),
        in_specs=[
            pl.BlockSpec((_TN, 1), lambda i: (i, 0)),
            pl.BlockSpec((_TN, H), lambda i: (i, 0)),
            pl.BlockSpec((128, H), lambda i: (0, 0)),
        ],
        out_specs=[
            pl.BlockSpec((_TN, HH), lambda i: (i, 0)),
            pl.BlockSpec((_TN, HH), lambda i: (i, 0)),
        ],
        out_shape=[
            jax.ShapeDtypeStruct((N, HH), _f32),
            jax.ShapeDtypeStruct((N, HH), _f32),
        ],
    )(an2, node_features, emb_pad)


def _filter_body(d_ref, w1_ref, b1_ref, w2_ref, b2_ref, w3_ref, b3_ref,
                 lo_ref, hi_ref):
    d = d_ref[...]
    centers = lax.broadcasted_iota(_i32, (1, NRBF), 1).astype(_f32) * (
        CUTOFF / (NRBF - 1))
    rbf = jnp.exp(-GAMMA * (d - centers) ** 2)
    u = d * (math.pi / CUTOFF)
    t = u * u
    cosu = 1.0 + t * (-1.0 / 2 + t * (1.0 / 24 + t * (-1.0 / 720 + t * (
        1.0 / 40320 + t * (-1.0 / 3628800 + t * (1.0 / 479001600))))))
    cf = 0.5 * (cosu + 1.0)
    cf = cf * (d < CUTOFF).astype(_f32)
    rbf = rbf * cf
    bf16 = jnp.bfloat16
    x = jnp.maximum(jnp.dot(rbf.astype(bf16), w1_ref[...].astype(bf16),
                            preferred_element_type=_f32) + b1_ref[...], 0.0)
    x = jnp.maximum(jnp.dot(x.astype(bf16), w2_ref[...].astype(bf16),
                            preferred_element_type=_f32) + b2_ref[...], 0.0)
    w = jnp.dot(x.astype(bf16), w3_ref[...].astype(bf16),
                preferred_element_type=_f32) + b3_ref[...]
    lo_ref[...] = w[:, :HH]
    hi_ref[...] = w[:, HH:]


def _filter_call(d, blk):
    grid = E // _TE
    full = lambda shape: pl.BlockSpec(shape, lambda i: (0, 0))
    return pl.pallas_call(
        _filter_body,
        grid=(grid,),
        in_specs=[
            pl.BlockSpec((_TE, 1), lambda i: (i, 0)),
            full((NRBF, NF)), full((1, NF)),
            full((NF, NF)), full((1, NF)),
            full((NF, H)), full((1, H)),
        ],
        out_specs=[
            pl.BlockSpec((_TE, HH), lambda i: (i, 0)),
            pl.BlockSpec((_TE, HH), lambda i: (i, 0)),
        ],
        out_shape=[
            jax.ShapeDtypeStruct((E, HH), _f32),
            jax.ShapeDtypeStruct((E, HH), _f32),
        ],
    )(d, blk['fw1'], blk['fb1'].reshape(1, NF),
      blk['fw2'], blk['fb2'].reshape(1, NF),
      blk['fw3'], blk['fb3'].reshape(1, H))


def _update_body(hlo_ref, hhi_ref, alo_ref, ahi_ref, cnt_ref,
                 w1_ref, b1_ref, w2_ref, b2_ref, g_ref, b_ref,
                 olo_ref, ohi_ref):
    h = jnp.concatenate([hlo_ref[...], hhi_ref[...]], axis=1)
    rc = 1.0 / jnp.maximum(cnt_ref[...][:, :1], 1.0)
    agg = jnp.concatenate([alo_ref[...], ahi_ref[...]], axis=1) * rc
    comb = jnp.concatenate([h, agg], axis=1)
    x = jnp.maximum(jnp.dot(comb, w1_ref[...], preferred_element_type=_f32)
                    + b1_ref[...], 0.0)
    hn = jnp.dot(x, w2_ref[...], preferred_element_type=_f32) + b2_ref[...]
    mu = jnp.mean(hn, axis=1, keepdims=True)
    var = jnp.mean((hn - mu) ** 2, axis=1, keepdims=True)
    hn = (hn - mu) / jnp.sqrt(var + 1e-5) * g_ref[...] + b_ref[...]
    out = h + hn
    olo_ref[...] = out[:, :HH]
    ohi_ref[...] = out[:, HH:]


def _update_call(h_lo, h_hi, agg_lo, agg_hi, cnt16, blk):
    grid = N // _TN
    half = pl.BlockSpec((_TN, HH), lambda i: (i, 0))
    full = lambda shape: pl.BlockSpec(shape, lambda i: (0, 0))
    return pl.pallas_call(
        _update_body,
        grid=(grid,),
        in_specs=[
            half, half, half, half,
            pl.BlockSpec((_TN, 16), lambda i: (i, 0)),
            full((2 * H, H)), full((1, H)),
            full((H, H)), full((1, H)),
            full((1, H)), full((1, H)),
        ],
        out_specs=[half, half],
        out_shape=[
            jax.ShapeDtypeStruct((N, HH), _f32),
            jax.ShapeDtypeStruct((N, HH), _f32),
        ],
    )(h_lo, h_hi, agg_lo, agg_hi, cnt16,
      blk['uw1'], blk['ub1'].reshape(1, H),
      blk['uw2'], blk['ub2'].reshape(1, H),
      blk['ln_g'].reshape(1, H), blk['ln_b'].reshape(1, H))


def _pool_body(b_ref, hlo_ref, hhi_ref,
               ew1_ref, eb1_ref, ew2_ref, eb2_ref,
               sw1_ref, sb1_ref, sw2_ref, sb2_ref,
               en_ref, sel_ref, acc_ref, cnt_ref):
    i = pl.program_id(0)

    @pl.when(i == 0)
    def _init():
        acc_ref[...] = jnp.zeros_like(acc_ref)
        cnt_ref[...] = jnp.zeros_like(cnt_ref)

    b = b_ref[...]
    oh = (b == lax.broadcasted_iota(_i32, (1, NG), 1)).astype(_f32)
    h = jnp.concatenate([hlo_ref[...], hhi_ref[...]], axis=1)
    dn = (((0,), (0,)), ((), ()))
    acc_ref[...] += lax.dot_general(oh, h, dn, preferred_element_type=_f32)
    cnt_ref[...] += lax.dot_general(oh, jnp.ones((oh.shape[0], 1), _f32), dn,
                                    preferred_element_type=_f32)

    @pl.when(i == pl.num_programs(0) - 1)
    def _final():
        pooled = acc_ref[...] / jnp.maximum(cnt_ref[...], 1.0)
        xe = jnp.maximum(jnp.dot(pooled, ew1_ref[...],
                                 preferred_element_type=_f32) + eb1_ref[...], 0.0)
        en_ref[...] = jnp.dot(xe, ew2_ref[...],
                              preferred_element_type=_f32) + eb2_ref[...]
        xs = jnp.maximum(jnp.dot(pooled, sw1_ref[...],
                                 preferred_element_type=_f32) + sb1_ref[...], 0.0)
        logits = jnp.dot(xs, sw2_ref[...],
                         preferred_element_type=_f32) + sb2_ref[...]
        z = logits - jnp.max(logits, axis=1, keepdims=True)
        ez = jnp.exp(z)
        sel_ref[...] = ez / jnp.sum(ez, axis=1, keepdims=True)


def _pool_call(b2, h_lo, h_hi, params):
    grid = N // _TN
    full = lambda shape: pl.BlockSpec(shape, lambda i: (0, 0))
    return pl.pallas_call(
        _pool_body,
        grid=(grid,),
        in_specs=[
            pl.BlockSpec((_TN, 1), lambda i: (i, 0)),
            pl.BlockSpec((_TN, HH), lambda i: (i, 0)),
            pl.BlockSpec((_TN, HH), lambda i: (i, 0)),
            full((H, H // 2)), full((1, H // 2)),
            full((H // 2, 1)), full((1, 1)),
            full((H, H // 2)), full((1, H // 2)),
            full((H // 2, 4)), full((1, 4)),
        ],
        out_specs=[full((NG, 1)), full((NG, 4))],
        out_shape=[
            jax.ShapeDtypeStruct((NG, 1), _f32),
            jax.ShapeDtypeStruct((NG, 4), _f32),
        ],
        scratch_shapes=[
            pltpu.VMEM((NG, H), _f32),
            pltpu.VMEM((NG, 1), _f32),
        ],
    )(b2, h_lo, h_hi,
      params['ew1'], params['eb1'].reshape(1, H // 2),
      params['ew2'], params['eb2'].reshape(1, 1),
      params['sw1'], params['sb1'].reshape(1, H // 2),
      params['sw2'], params['sb2'].reshape(1, 4))


@functools.lru_cache(maxsize=1)
def _sc_mesh():
    return plsc.VectorSubcoreMesh(core_axis_name="c", subcore_axis_name="s",
                                  num_cores=2, num_subcores=NSUB)


def _zero_rows(zbuf, width):
    def fill(r, _):
        zbuf[r, 0:16] = jnp.zeros((16,), _f32)
        if width > 16:
            zbuf[r, 16:32] = jnp.zeros((16,), _f32)
        return 0
    lax.fori_loop(0, ZROWS, fill, 0)


def _nchunks(s):
    return CH_PER_SUB + (s < CH_EXTRA).astype(_i32)


def _cnt_body(dst_hbm, out_hbm, idx_buf, ones_buf, zbuf, cnt_sh):
    c = lax.axis_index("c")
    s = lax.axis_index("s")

    @pl.when(c == 0)
    def _run():
        _zero_rows(zbuf, 16)

        def fill(r, _):
            ones_buf[r, 0:16] = jnp.ones((16,), _f32)
            return 0
        lax.fori_loop(0, CH, fill, 0)

        def zcp(k, _):
            pltpu.sync_copy(zbuf,
                            cnt_sh.at[pl.ds(s * ROWS_PER_SUB + k * ZROWS, ZROWS)])
            return 0
        lax.fori_loop(0, ROWS_PER_SUB // ZROWS, zcp, 0)
        plsc.subcore_barrier()

        def chunk(k, _):
            base = (s + k * NSUB) * CH
            pltpu.sync_copy(dst_hbm.at[pl.ds(base, CH)], idx_buf.at[0])
            pltpu.sync_copy(ones_buf, cnt_sh.at[idx_buf.at[0]], add=True)
            return 0
        lax.fori_loop(0, _nchunks(s), chunk, 0)
        plsc.subcore_barrier()

        pltpu.sync_copy(cnt_sh.at[pl.ds(s * ROWS_PER_SUB, ROWS_PER_SUB)],
                        out_hbm.at[pl.ds(s * ROWS_PER_SUB, ROWS_PER_SUB)])


def _cnt_call(dst):
    return pl.kernel(
        _cnt_body,
        out_type=[jax.ShapeDtypeStruct((NPAD, 16), _f32)],
        mesh=_sc_mesh(),
        scratch_types=[
            pltpu.VMEM((1, CH), _i32),
            pltpu.VMEM((CH, 16), _f32),
            pltpu.VMEM((ZROWS, 16), _f32),
            pltpu.VMEM_SHARED((NPAD, 16), _f32),
        ],
        compiler_params=pltpu.CompilerParams(use_tc_tiling_on_sc=False),
    )(dst)[0]


def _msg_half(h_hbm, w_hbm, agg_hbm, src1, dst1,
              isrc2, idxd2, rows, wbuf, gsem, wsem, ssem, isem,
              agg_sh, s):
    def zfill(r, _):
        rows[r, 0:16] = jnp.zeros((16,), _f32)
        rows[r, 16:32] = jnp.zeros((16,), _f32)
        return 0
    lax.fori_loop(0, ZCP, zfill, 0)

    def zcp(k, _):
        pltpu.sync_copy(rows.at[pl.ds(0, ZCP)],
                        agg_sh.at[pl.ds(s * ROWS_PER_SUB + k * ZCP, ZCP)])
        return 0
    lax.fori_loop(0, ROWS_PER_SUB // ZCP, zcp, 0)
    plsc.subcore_barrier()

    def mult(n4):
        def mrow(q, _):
            for j in range(4):
                r = q * 4 + j
                rows[r, 0:16] = rows[r, 0:16] * wbuf[r, 0:16]
                rows[r, 16:32] = rows[r, 16:32] * wbuf[r, 16:32]
            return 0
        lax.fori_loop(0, n4, mrow, 0)

    start_c = CPS * s
    nc = jnp.minimum(jnp.maximum(NCHUNKS - start_c, 0), CPS)
    nsc = nc // SS

    def superchunk(p, _):
        c0 = start_c + SS * p
        ids = [pltpu.async_copy(src1.at[pl.ds((c0 + j) * CH, CH)],
                                isrc2.at[j], isem) for j in range(SS)]
        ids += [pltpu.async_copy(dst1.at[pl.ds((c0 + j) * CH, CH)],
                                 idxd2.at[j], isem) for j in range(SS)]
        for d in ids:
            d.wait()
        gds = [pltpu.async_copy(h_hbm.at[isrc2.at[j]],
                                rows.at[pl.ds(j * CH, CH)], gsem)
               for j in range(SS)]
        wd = pltpu.async_copy(w_hbm.at[pl.ds(c0 * CH, SS * CH)], wbuf, wsem)
        for d in gds:
            d.wait()
        wd.wait()
        mult(SS * CH // 4)
        sds = [pltpu.async_copy(rows.at[pl.ds(j * CH, CH)],
                                agg_sh.at[idxd2.at[j]], ssem, add=True)
               for j in range(SS)]
        for d in sds:
            d.wait()
        return 0
    lax.fori_loop(0, nsc, superchunk, 0)

    def single(r, _):
        base = (start_c + nsc * SS + r) * CH
        pltpu.sync_copy(src1.at[pl.ds(base, CH)], isrc2.at[0])
        pltpu.sync_copy(dst1.at[pl.ds(base, CH)], idxd2.at[0])
        pltpu.sync_copy(h_hbm.at[isrc2.at[0]], rows.at[pl.ds(0, CH)])
        pltpu.sync_copy(w_hbm.at[pl.ds(base, CH)], wbuf.at[pl.ds(0, CH)])
        mult(CH // 4)
        pltpu.sync_copy(rows.at[pl.ds(0, CH)], agg_sh.at[idxd2.at[0]],
                        add=True)
        return 0
    lax.fori_loop(0, nc - nsc * SS, single, 0)

    plsc.subcore_barrier()

    pltpu.sync_copy(agg_sh.at[pl.ds(s * ROWS_PER_SUB, ROWS_PER_SUB)],
                    agg_hbm.at[pl.ds(s * ROWS_PER_SUB, ROWS_PER_SUB)])


def _msg_body(hlo_hbm, hhi_hbm, wlo_hbm, whi_hbm,
              src1_hbm, dst1_hbm, cnt_hbm,
              alo_hbm, ahi_hbm,
              isrc2, idxd2, rows, wbuf, touch, gsem, wsem, ssem, isem,
              agg_sh):
    c = lax.axis_index("c")
    s = lax.axis_index("s")
    pltpu.sync_copy(cnt_hbm.at[pl.ds(0, 1)], touch)

    @pl.when(c == 0)
    def _lo():
        _msg_half(hlo_hbm, wlo_hbm, alo_hbm, src1_hbm, dst1_hbm,
                  isrc2, idxd2, rows, wbuf,
                  gsem, wsem, ssem, isem, agg_sh, s)

    @pl.when(c == 1)
    def _hi():
        _msg_half(hhi_hbm, whi_hbm, ahi_hbm, src1_hbm, dst1_hbm,
                  isrc2, idxd2, rows, wbuf,
                  gsem, wsem, ssem, isem, agg_sh, s)


def _msg_call(h_lo, h_hi, w_lo, w_hi, src, dst, cnt16):
    return pl.kernel(
        _msg_body,
        out_type=[
            jax.ShapeDtypeStruct((NPAD, HH), _f32),
            jax.ShapeDtypeStruct((NPAD, HH), _f32),
        ],
        mesh=_sc_mesh(),
        scratch_types=[
            pltpu.VMEM((SS, CH), _i32),
            pltpu.VMEM((SS, CH), _i32),
            pltpu.VMEM((SS * CH, HH), _f32),
            pltpu.VMEM((SS * CH, HH), _f32),
            pltpu.VMEM((1, 16), _f32),
            pltpu.SemaphoreType.DMA,
            pltpu.SemaphoreType.DMA,
            pltpu.SemaphoreType.DMA,
            pltpu.SemaphoreType.DMA,
            pltpu.VMEM_SHARED((NPAD, HH), _f32),
        ],
        compiler_params=pltpu.CompilerParams(use_tc_tiling_on_sc=False),
    )(h_lo, h_hi, w_lo, w_hi, src, dst, cnt16)


def kernel(node_features, edge_index, edge_dist, batch, atomic_numbers, params):
    src = edge_index[0]
    dst = edge_index[1]
    an2 = atomic_numbers.reshape(N, 1).astype(_i32)
    b2 = batch.reshape(N, 1).astype(_i32)
    emb_pad = jnp.pad(params['embed'], ((0, 128 - params['embed'].shape[0]),
                                        (0, 0)))

    h_lo, h_hi = _embed_call(an2, node_features, emb_pad)
    cnt16 = _cnt_call(dst)

    for blk in params['blocks']:
        w_lo, w_hi = _filter_call(edge_dist, blk)
        agg_lo, agg_hi = _msg_call(h_lo, h_hi, w_lo, w_hi, src, dst, cnt16)
        h_lo, h_hi = _update_call(h_lo, h_hi, agg_lo, agg_hi, cnt16, blk)

    energy, selectivity = _pool_call(b2, h_lo, h_hi, params)
    h = jnp.concatenate([h_lo, h_hi], axis=1)
    return energy, selectivity, h


# speedup vs baseline: 1.9769x; 1.0004x over previous
import functools
import math

import jax
import jax.numpy as jnp
from jax import lax
from jax.experimental import pallas as pl
from jax.experimental.pallas import tpu as pltpu
from jax.experimental.pallas import tpu_sc as plsc

N = 50000
E = 800000
H = 64
HH = 32
NRBF = 64
NF = 128
NG = 64
CUTOFF = 6.0
GAMMA = 10.0

NSUB = 16
CH = 128
NCHUNKS = E // CH
CH_PER_SUB = NCHUNKS // NSUB
CH_EXTRA = NCHUNKS - CH_PER_SUB * NSUB
CPS = 392
SS = 3
ZCP = 196
NPAD = 50176
ROWS_PER_SUB = NPAD // NSUB
ZROWS = 392

_f32 = jnp.float32
_i32 = jnp.int32


_TN = 5000
_TE = 6400


def _embed_body(an_ref, nf_ref, emb_ref, lo_ref, hi_ref):
    a = an_ref[...]
    oh = (a == lax.broadcasted_iota(_i32, (1, 128), 1)).astype(_f32)
    h = jnp.dot(oh, emb_ref[...], preferred_element_type=_f32) + nf_ref[...]
    lo_ref[...] = h[:, :HH]
    hi_ref[...] = h[:, HH:]


def _embed_call(an2, node_features, emb_pad):
    grid = N // _TN
    return pl.pallas_call(
        _embed_body,
        grid=(grid,),
        in_specs=[
            pl.BlockSpec((_TN, 1), lambda i: (i, 0)),
            pl.BlockSpec((_TN, H), lambda i: (i, 0)),
            pl.BlockSpec((128, H), lambda i: (0, 0)),
        ],
        out_specs=[
            pl.BlockSpec((_TN, HH), lambda i: (i, 0)),
            pl.BlockSpec((_TN, HH), lambda i: (i, 0)),
        ],
        out_shape=[
            jax.ShapeDtypeStruct((N, HH), _f32),
            jax.ShapeDtypeStruct((N, HH), _f32),
        ],
    )(an2, node_features, emb_pad)


def _filter_body(d_ref, w1_ref, b1_ref, w2_ref, b2_ref, w3_ref, b3_ref,
                 lo_ref, hi_ref):
    d = d_ref[...]
    centers = lax.broadcasted_iota(_i32, (1, NRBF), 1).astype(_f32) * (
        CUTOFF / (NRBF - 1))
    rbf = jnp.exp(-GAMMA * (d - centers) ** 2)
    u = d * (math.pi / CUTOFF)
    t = u * u
    cosu = 1.0 + t * (-1.0 / 2 + t * (1.0 / 24 + t * (-1.0 / 720 + t * (
        1.0 / 40320 + t * (-1.0 / 3628800 + t * (1.0 / 479001600))))))
    cf = 0.5 * (cosu + 1.0)
    cf = cf * (d < CUTOFF).astype(_f32)
    rbf = rbf * cf
    bf16 = jnp.bfloat16
    x = jnp.maximum(jnp.dot(rbf.astype(bf16), w1_ref[...].astype(bf16),
                            preferred_element_type=_f32) + b1_ref[...], 0.0)
    x = jnp.maximum(jnp.dot(x.astype(bf16), w2_ref[...].astype(bf16),
                            preferred_element_type=_f32) + b2_ref[...], 0.0)
    w = jnp.dot(x.astype(bf16), w3_ref[...].astype(bf16),
                preferred_element_type=_f32) + b3_ref[...]
    lo_ref[...] = w[:, :HH]
    hi_ref[...] = w[:, HH:]


def _filter_call(d, blk):
    grid = E // _TE
    full = lambda shape: pl.BlockSpec(shape, lambda i: (0, 0))
    return pl.pallas_call(
        _filter_body,
        grid=(grid,),
        in_specs=[
            pl.BlockSpec((_TE, 1), lambda i: (i, 0)),
            full((NRBF, NF)), full((1, NF)),
            full((NF, NF)), full((1, NF)),
            full((NF, H)), full((1, H)),
        ],
        out_specs=[
            pl.BlockSpec((_TE, HH), lambda i: (i, 0)),
            pl.BlockSpec((_TE, HH), lambda i: (i, 0)),
        ],
        out_shape=[
            jax.ShapeDtypeStruct((E, HH), _f32),
            jax.ShapeDtypeStruct((E, HH), _f32),
        ],
    )(d, blk['fw1'], blk['fb1'].reshape(1, NF),
      blk['fw2'], blk['fb2'].reshape(1, NF),
      blk['fw3'], blk['fb3'].reshape(1, H))


def _update_body(hlo_ref, hhi_ref, alo_ref, ahi_ref, cnt_ref,
                 w1_ref, b1_ref, w2_ref, b2_ref, g_ref, b_ref,
                 olo_ref, ohi_ref):
    h = jnp.concatenate([hlo_ref[...], hhi_ref[...]], axis=1)
    rc = 1.0 / jnp.maximum(cnt_ref[...][:, :1], 1.0)
    agg = jnp.concatenate([alo_ref[...], ahi_ref[...]], axis=1) * rc
    comb = jnp.concatenate([h, agg], axis=1)
    x = jnp.maximum(jnp.dot(comb, w1_ref[...], preferred_element_type=_f32)
                    + b1_ref[...], 0.0)
    hn = jnp.dot(x, w2_ref[...], preferred_element_type=_f32) + b2_ref[...]
    mu = jnp.mean(hn, axis=1, keepdims=True)
    var = jnp.mean((hn - mu) ** 2, axis=1, keepdims=True)
    hn = (hn - mu) / jnp.sqrt(var + 1e-5) * g_ref[...] + b_ref[...]
    out = h + hn
    olo_ref[...] = out[:, :HH]
    ohi_ref[...] = out[:, HH:]


def _update_call(h_lo, h_hi, agg_lo, agg_hi, cnt16, blk):
    grid = N // _TN
    half = pl.BlockSpec((_TN, HH), lambda i: (i, 0))
    full = lambda shape: pl.BlockSpec(shape, lambda i: (0, 0))
    return pl.pallas_call(
        _update_body,
        grid=(grid,),
        in_specs=[
            half, half, half, half,
            pl.BlockSpec((_TN, 16), lambda i: (i, 0)),
            full((2 * H, H)), full((1, H)),
            full((H, H)), full((1, H)),
            full((1, H)), full((1, H)),
        ],
        out_specs=[half, half],
        out_shape=[
            jax.ShapeDtypeStruct((N, HH), _f32),
            jax.ShapeDtypeStruct((N, HH), _f32),
        ],
    )(h_lo, h_hi, agg_lo, agg_hi, cnt16,
      blk['uw1'], blk['ub1'].reshape(1, H),
      blk['uw2'], blk['ub2'].reshape(1, H),
      blk['ln_g'].reshape(1, H), blk['ln_b'].reshape(1, H))


def _pool_body(b_ref, hlo_ref, hhi_ref,
               ew1_ref, eb1_ref, ew2_ref, eb2_ref,
               sw1_ref, sb1_ref, sw2_ref, sb2_ref,
               en_ref, sel_ref, acc_ref, cnt_ref):
    i = pl.program_id(0)

    @pl.when(i == 0)
    def _init():
        acc_ref[...] = jnp.zeros_like(acc_ref)
        cnt_ref[...] = jnp.zeros_like(cnt_ref)

    b = b_ref[...]
    oh = (b == lax.broadcasted_iota(_i32, (1, NG), 1)).astype(_f32)
    h = jnp.concatenate([hlo_ref[...], hhi_ref[...]], axis=1)
    dn = (((0,), (0,)), ((), ()))
    acc_ref[...] += lax.dot_general(oh, h, dn, preferred_element_type=_f32)
    cnt_ref[...] += lax.dot_general(oh, jnp.ones((oh.shape[0], 1), _f32), dn,
                                    preferred_element_type=_f32)

    @pl.when(i == pl.num_programs(0) - 1)
    def _final():
        pooled = acc_ref[...] / jnp.maximum(cnt_ref[...], 1.0)
        xe = jnp.maximum(jnp.dot(pooled, ew1_ref[...],
                                 preferred_element_type=_f32) + eb1_ref[...], 0.0)
        en_ref[...] = jnp.dot(xe, ew2_ref[...],
                              preferred_element_type=_f32) + eb2_ref[...]
        xs = jnp.maximum(jnp.dot(pooled, sw1_ref[...],
                                 preferred_element_type=_f32) + sb1_ref[...], 0.0)
        logits = jnp.dot(xs, sw2_ref[...],
                         preferred_element_type=_f32) + sb2_ref[...]
        z = logits - jnp.max(logits, axis=1, keepdims=True)
        ez = jnp.exp(z)
        sel_ref[...] = ez / jnp.sum(ez, axis=1, keepdims=True)


def _pool_call(b2, h_lo, h_hi, params):
    grid = N // _TN
    full = lambda shape: pl.BlockSpec(shape, lambda i: (0, 0))
    return pl.pallas_call(
        _pool_body,
        grid=(grid,),
        in_specs=[
            pl.BlockSpec((_TN, 1), lambda i: (i, 0)),
            pl.BlockSpec((_TN, HH), lambda i: (i, 0)),
            pl.BlockSpec((_TN, HH), lambda i: (i, 0)),
            full((H, H // 2)), full((1, H // 2)),
            full((H // 2, 1)), full((1, 1)),
            full((H, H // 2)), full((1, H // 2)),
            full((H // 2, 4)), full((1, 4)),
        ],
        out_specs=[full((NG, 1)), full((NG, 4))],
        out_shape=[
            jax.ShapeDtypeStruct((NG, 1), _f32),
            jax.ShapeDtypeStruct((NG, 4), _f32),
        ],
        scratch_shapes=[
            pltpu.VMEM((NG, H), _f32),
            pltpu.VMEM((NG, 1), _f32),
        ],
    )(b2, h_lo, h_hi,
      params['ew1'], params['eb1'].reshape(1, H // 2),
      params['ew2'], params['eb2'].reshape(1, 1),
      params['sw1'], params['sb1'].reshape(1, H // 2),
      params['sw2'], params['sb2'].reshape(1, 4))


@functools.lru_cache(maxsize=1)
def _sc_mesh():
    return plsc.VectorSubcoreMesh(core_axis_name="c", subcore_axis_name="s",
                                  num_cores=2, num_subcores=NSUB)


def _zero_rows(zbuf, width):
    def fill(r, _):
        zbuf[r, 0:16] = jnp.zeros((16,), _f32)
        if width > 16:
            zbuf[r, 16:32] = jnp.zeros((16,), _f32)
        return 0
    lax.fori_loop(0, ZROWS, fill, 0)


def _nchunks(s):
    return CH_PER_SUB + (s < CH_EXTRA).astype(_i32)


def _cnt_body(dst_hbm, out_hbm, idx_buf, ones_buf, zbuf, cnt_sh):
    c = lax.axis_index("c")
    s = lax.axis_index("s")

    @pl.when(c == 0)
    def _run():
        _zero_rows(zbuf, 16)

        def fill(r, _):
            ones_buf[r, 0:16] = jnp.ones((16,), _f32)
            return 0
        lax.fori_loop(0, CH, fill, 0)

        def zcp(k, _):
            pltpu.sync_copy(zbuf,
                            cnt_sh.at[pl.ds(s * ROWS_PER_SUB + k * ZROWS, ZROWS)])
            return 0
        lax.fori_loop(0, ROWS_PER_SUB // ZROWS, zcp, 0)
        plsc.subcore_barrier()

        def chunk(k, _):
            base = (s + k * NSUB) * CH
            pltpu.sync_copy(dst_hbm.at[pl.ds(base, CH)], idx_buf.at[0])
            pltpu.sync_copy(ones_buf, cnt_sh.at[idx_buf.at[0]], add=True)
            return 0
        lax.fori_loop(0, _nchunks(s), chunk, 0)
        plsc.subcore_barrier()

        pltpu.sync_copy(cnt_sh.at[pl.ds(s * ROWS_PER_SUB, ROWS_PER_SUB)],
                        out_hbm.at[pl.ds(s * ROWS_PER_SUB, ROWS_PER_SUB)])


def _cnt_call(dst):
    return pl.kernel(
        _cnt_body,
        out_type=[jax.ShapeDtypeStruct((NPAD, 16), _f32)],
        mesh=_sc_mesh(),
        scratch_types=[
            pltpu.VMEM((1, CH), _i32),
            pltpu.VMEM((CH, 16), _f32),
            pltpu.VMEM((ZROWS, 16), _f32),
            pltpu.VMEM_SHARED((NPAD, 16), _f32),
        ],
        compiler_params=pltpu.CompilerParams(use_tc_tiling_on_sc=False),
    )(dst)[0]


def _msg_half(h_hbm, w_hbm, agg_hbm, src1, dst1,
              isrc2, idxd2, rows, wbuf, gsem, wsem, ssem, isem,
              agg_sh, s):
    def zfill(r, _):
        rows[r, 0:16] = jnp.zeros((16,), _f32)
        rows[r, 16:32] = jnp.zeros((16,), _f32)
        return 0
    lax.fori_loop(0, ZCP, zfill, 0)

    def zcp(k, _):
        pltpu.sync_copy(rows.at[pl.ds(0, ZCP)],
                        agg_sh.at[pl.ds(s * ROWS_PER_SUB + k * ZCP, ZCP)])
        return 0
    lax.fori_loop(0, ROWS_PER_SUB // ZCP, zcp, 0)
    plsc.subcore_barrier()

    def mult(n8):
        def mrow(q, _):
            for j in range(8):
                r = q * 8 + j
                rows[r, 0:16] = rows[r, 0:16] * wbuf[r, 0:16]
                rows[r, 16:32] = rows[r, 16:32] * wbuf[r, 16:32]
            return 0
        lax.fori_loop(0, n8, mrow, 0)

    start_c = CPS * s
    nc = jnp.minimum(jnp.maximum(NCHUNKS - start_c, 0), CPS)
    nsc = nc // SS

    def superchunk(p, _):
        c0 = start_c + SS * p
        ids = [pltpu.async_copy(src1.at[pl.ds((c0 + j) * CH, CH)],
                                isrc2.at[j], isem) for j in range(SS)]
        ids += [pltpu.async_copy(dst1.at[pl.ds((c0 + j) * CH, CH)],
                                 idxd2.at[j], isem) for j in range(SS)]
        for d in ids:
            d.wait()
        gds = [pltpu.async_copy(h_hbm.at[isrc2.at[j]],
                                rows.at[pl.ds(j * CH, CH)], gsem)
               for j in range(SS)]
        wd = pltpu.async_copy(w_hbm.at[pl.ds(c0 * CH, SS * CH)], wbuf, wsem)
        for d in gds:
            d.wait()
        wd.wait()
        mult(SS * CH // 8)
        sds = [pltpu.async_copy(rows.at[pl.ds(j * CH, CH)],
                                agg_sh.at[idxd2.at[j]], ssem, add=True)
               for j in range(SS)]
        for d in sds:
            d.wait()
        return 0
    lax.fori_loop(0, nsc, superchunk, 0)

    def single(r, _):
        base = (start_c + nsc * SS + r) * CH
        pltpu.sync_copy(src1.at[pl.ds(base, CH)], isrc2.at[0])
        pltpu.sync_copy(dst1.at[pl.ds(base, CH)], idxd2.at[0])
        pltpu.sync_copy(h_hbm.at[isrc2.at[0]], rows.at[pl.ds(0, CH)])
        pltpu.sync_copy(w_hbm.at[pl.ds(base, CH)], wbuf.at[pl.ds(0, CH)])
        mult(CH // 8)
        pltpu.sync_copy(rows.at[pl.ds(0, CH)], agg_sh.at[idxd2.at[0]],
                        add=True)
        return 0
    lax.fori_loop(0, nc - nsc * SS, single, 0)

    plsc.subcore_barrier()

    pltpu.sync_copy(agg_sh.at[pl.ds(s * ROWS_PER_SUB, ROWS_PER_SUB)],
                    agg_hbm.at[pl.ds(s * ROWS_PER_SUB, ROWS_PER_SUB)])


def _msg_body(hlo_hbm, hhi_hbm, wlo_hbm, whi_hbm,
              src1_hbm, dst1_hbm, cnt_hbm,
              alo_hbm, ahi_hbm,
              isrc2, idxd2, rows, wbuf, touch, gsem, wsem, ssem, isem,
              agg_sh):
    c = lax.axis_index("c")
    s = lax.axis_index("s")
    pltpu.sync_copy(cnt_hbm.at[pl.ds(0, 1)], touch)

    @pl.when(c == 0)
    def _lo():
        _msg_half(hlo_hbm, wlo_hbm, alo_hbm, src1_hbm, dst1_hbm,
                  isrc2, idxd2, rows, wbuf,
                  gsem, wsem, ssem, isem, agg_sh, s)

    @pl.when(c == 1)
    def _hi():
        _msg_half(hhi_hbm, whi_hbm, ahi_hbm, src1_hbm, dst1_hbm,
                  isrc2, idxd2, rows, wbuf,
                  gsem, wsem, ssem, isem, agg_sh, s)


def _msg_call(h_lo, h_hi, w_lo, w_hi, src, dst, cnt16):
    return pl.kernel(
        _msg_body,
        out_type=[
            jax.ShapeDtypeStruct((NPAD, HH), _f32),
            jax.ShapeDtypeStruct((NPAD, HH), _f32),
        ],
        mesh=_sc_mesh(),
        scratch_types=[
            pltpu.VMEM((SS, CH), _i32),
            pltpu.VMEM((SS, CH), _i32),
            pltpu.VMEM((SS * CH, HH), _f32),
            pltpu.VMEM((SS * CH, HH), _f32),
            pltpu.VMEM((1, 16), _f32),
            pltpu.SemaphoreType.DMA,
            pltpu.SemaphoreType.DMA,
            pltpu.SemaphoreType.DMA,
            pltpu.SemaphoreType.DMA,
            pltpu.VMEM_SHARED((NPAD, HH), _f32),
        ],
        compiler_params=pltpu.CompilerParams(use_tc_tiling_on_sc=False),
    )(h_lo, h_hi, w_lo, w_hi, src, dst, cnt16)


def kernel(node_features, edge_index, edge_dist, batch, atomic_numbers, params):
    src = edge_index[0]
    dst = edge_index[1]
    an2 = atomic_numbers.reshape(N, 1).astype(_i32)
    b2 = batch.reshape(N, 1).astype(_i32)
    emb_pad = jnp.pad(params['embed'], ((0, 128 - params['embed'].shape[0]),
                                        (0, 0)))

    h_lo, h_hi = _embed_call(an2, node_features, emb_pad)
    cnt16 = _cnt_call(dst)

    for blk in params['blocks']:
        w_lo, w_hi = _filter_call(edge_dist, blk)
        agg_lo, agg_hi = _msg_call(h_lo, h_hi, w_lo, w_hi, src, dst, cnt16)
        h_lo, h_hi = _update_call(h_lo, h_hi, agg_lo, agg_hi, cnt16, blk)

    energy, selectivity = _pool_call(b2, h_lo, h_hi, params)
    h = jnp.concatenate([h_lo, h_hi], axis=1)
    return energy, selectivity, h
